# Initial kernel scaffold; baseline (speedup 1.0000x reference)
import jax
import jax.numpy as jnp
from jax.experimental import pallas as pl


def kernel(last_hidden, attention_mask, variety_ids, W_down, b_down, W_up, b_up, W_c, b_c):
    raise NotImplementedError("write your pallas kernel here")


# speedup vs baseline: 2.3219x; 2.3219x over previous
import jax
import jax.numpy as jnp
from jax.experimental import pallas as pl
from jax.experimental.pallas import tpu as pltpu

B, T, H, A, E, L = 128, 512, 1024, 128, 16, 1000
L_PAD = 1024


def _adapter_head_kernel(x_ref, vids_ref, Wd_ref, bd_ref, Wu_ref, bu_ref,
                         Wc_ref, bc_ref, out_ref):
    x = x_ref[...]
    vids = vids_ref[...]
    up = jnp.zeros((B, H), dtype=jnp.float32)
    for e in range(E):
        m = (vids == e).astype(jnp.float32)
        h = jnp.dot(x, Wd_ref[e], preferred_element_type=jnp.float32)
        h = jnp.maximum(h + bd_ref[e], 0.0) * m
        up = up + jnp.dot(h, Wu_ref[e], preferred_element_type=jnp.float32)
        up = up + m * bu_ref[e]
    out = x + up
    logits = jnp.dot(out, Wc_ref[...], preferred_element_type=jnp.float32)
    out_ref[...] = logits + bc_ref[...]


def kernel(last_hidden, attention_mask, variety_ids, W_down, b_down, W_up,
           b_up, W_c, b_c):
    x = last_hidden[:, 0, :]
    vids = variety_ids.astype(jnp.int32).reshape(B, 1)
    Wc_p = jnp.zeros((H, L_PAD), dtype=jnp.float32).at[:, :L].set(W_c)
    bc_p = jnp.zeros((1, L_PAD), dtype=jnp.float32).at[0, :L].set(b_c)

    logits_p = pl.pallas_call(
        _adapter_head_kernel,
        out_shape=jax.ShapeDtypeStruct((B, L_PAD), jnp.float32),
    )(x, vids, W_down, b_down.reshape(E, 1, A), W_up, b_up.reshape(E, 1, H),
      Wc_p, bc_p)
    return logits_p[:, :L]
